# baseline (speedup 1.0000x reference)
import functools

import jax
import jax.numpy as jnp
from jax import lax
from jax.experimental import pallas as pl
from jax.experimental.pallas import tpu as pltpu
from jax.experimental.pallas import tpu_sc as plsc

_NC = 2
_NS = 16
_NW = _NC * _NS
_B = 4096
_D = 128
_BPW = _B // _NW

_mesh = plsc.VectorSubcoreMesh(core_axis_name="c", subcore_axis_name="s")


_H = _BPW // 2


@functools.partial(
    pl.kernel,
    mesh=_mesh,
    out_type=jax.ShapeDtypeStruct((_B, _D), jnp.float32),
    scratch_types=[
        pltpu.VMEM((_BPW,), jnp.int32),
        pltpu.VMEM((_BPW, _D), jnp.float32),
        pltpu.SemaphoreType.DMA,
        pltpu.SemaphoreType.DMA,
        pltpu.SemaphoreType.DMA,
    ],
)
def _gather_kernel(idx_hbm, table_hbm, out_hbm, idx_v, rows_v, sg0, sg1, sw):
    wid = lax.axis_index("s") * _NC + lax.axis_index("c")
    base = wid * _BPW
    pltpu.sync_copy(idx_hbm.at[pl.ds(base, _BPW)], idx_v)
    g0 = pltpu.async_copy(
        table_hbm.at[idx_v.at[pl.ds(0, _H)]], rows_v.at[pl.ds(0, _H)], sg0)
    g0.wait()
    g1 = pltpu.async_copy(
        table_hbm.at[idx_v.at[pl.ds(_H, _H)]], rows_v.at[pl.ds(_H, _H)], sg1)
    w0 = pltpu.async_copy(
        rows_v.at[pl.ds(0, _H)], out_hbm.at[pl.ds(base, _H)], sw)
    g1.wait()
    w1 = pltpu.async_copy(
        rows_v.at[pl.ds(_H, _H)], out_hbm.at[pl.ds(base + _H, _H)], sw)
    w0.wait()
    w1.wait()


def kernel(time_slots, preference):
    return _gather_kernel(time_slots.astype(jnp.int32), preference)


# speedup vs baseline: 1.0350x; 1.0350x over previous
import functools

import jax
import jax.numpy as jnp
from jax import lax
from jax.experimental import pallas as pl
from jax.experimental.pallas import tpu as pltpu
from jax.experimental.pallas import tpu_sc as plsc

_NC = 2
_NS = 16
_NW = _NC * _NS
_B = 4096
_D = 128
_BPW = _B // _NW

_mesh = plsc.VectorSubcoreMesh(core_axis_name="c", subcore_axis_name="s")


_H = _BPW // 2


@functools.partial(
    pl.kernel,
    mesh=_mesh,
    out_type=jax.ShapeDtypeStruct((_B, _D), jnp.float32),
    scratch_types=[
        pltpu.VMEM((_BPW,), jnp.int32),
        pltpu.VMEM((_BPW, _D), jnp.float32),
        pltpu.SemaphoreType.DMA,
        pltpu.SemaphoreType.DMA,
        pltpu.SemaphoreType.DMA,
    ],
)
def _gather_kernel(idx_hbm, table_hbm, out_hbm, idx_v, rows_v, sg0, sg1, sw):
    wid = lax.axis_index("s") * _NC + lax.axis_index("c")
    base = wid * _BPW
    pltpu.sync_copy(idx_hbm.at[pl.ds(base, _BPW)], idx_v)
    g0 = pltpu.async_copy(
        table_hbm.at[idx_v.at[pl.ds(0, _H)]], rows_v.at[pl.ds(0, _H)], sg0)
    g1 = pltpu.async_copy(
        table_hbm.at[idx_v.at[pl.ds(_H, _H)]], rows_v.at[pl.ds(_H, _H)], sg1)
    g0.wait()
    w0 = pltpu.async_copy(
        rows_v.at[pl.ds(0, _H)], out_hbm.at[pl.ds(base, _H)], sw)
    g1.wait()
    w1 = pltpu.async_copy(
        rows_v.at[pl.ds(_H, _H)], out_hbm.at[pl.ds(base + _H, _H)], sw)
    w0.wait()
    w1.wait()


def kernel(time_slots, preference):
    return _gather_kernel(time_slots.astype(jnp.int32), preference)
